# Initial kernel scaffold; baseline (speedup 1.0000x reference)
import jax
import jax.numpy as jnp
from jax.experimental import pallas as pl


def kernel(h, W, expert_embeddings, tau):
    raise NotImplementedError("write your pallas kernel here")


# speedup vs baseline: 2.8957x; 2.8957x over previous
import functools

import jax
import jax.numpy as jnp
from jax.experimental import pallas as pl
from jax.experimental.pallas import tpu as pltpu

_NUM_TOK = 8192
_IN_DIM = 4096
_NUM_EXPERTS = 64
_D_E = 64
_TOP_K = 8
_BLK = 512


def _router_block(tau_ref, h_ref, w_ref, ee_ref, sg_ref, idx_ref, fg_ref):
    f32 = jnp.float32
    hp = jax.lax.dot_general(
        h_ref[...], w_ref[...], (((1,), (1,)), ((), ())),
        preferred_element_type=f32, precision=jax.lax.Precision.DEFAULT)
    nrm = jnp.sqrt(jnp.sum(hp * hp, axis=-1, keepdims=True))
    hn = hp / jnp.maximum(nrm, 1e-12)
    ee = ee_ref[...]
    een = ee / jnp.maximum(
        jnp.sqrt(jnp.sum(ee * ee, axis=-1, keepdims=True)), 1e-12)
    scores = jax.lax.dot_general(
        hn, een, (((1,), (1,)), ((), ())),
        preferred_element_type=f32, precision=jax.lax.Precision.DEFAULT)
    x = scores / tau_ref[0]
    m = jnp.max(x, axis=-1, keepdims=True)
    ex = jnp.exp(x - m)
    fg = ex / jnp.sum(ex, axis=-1, keepdims=True)
    fg_ref[...] = fg

    iota_e = jax.lax.broadcasted_iota(jnp.int32, fg.shape, 1)
    iota_k = jax.lax.broadcasted_iota(jnp.int32, (fg.shape[0], _TOP_K), 1)
    work = fg
    vals = jnp.zeros((fg.shape[0], _TOP_K), f32)
    idxs = jnp.zeros((fg.shape[0], _TOP_K), jnp.int32)
    for k in range(_TOP_K):
        v = jnp.max(work, axis=-1, keepdims=True)
        i = jnp.min(jnp.where(work == v, iota_e, _NUM_EXPERTS),
                    axis=-1, keepdims=True)
        vals = jnp.where(iota_k == k, v, vals)
        idxs = jnp.where(iota_k == k, i, idxs)
        work = jnp.where(iota_e == i, -1.0, work)
    idx_ref[...] = idxs

    ev = jnp.exp(vals - jax.lax.slice_in_dim(vals, 0, 1, axis=1))
    nt = ev / jnp.sum(ev, axis=-1, keepdims=True)

    sg = jnp.zeros(fg.shape, f32)
    for k in range(_TOP_K):
        sg = sg + jnp.where(
            iota_e == jax.lax.slice_in_dim(idxs, k, k + 1, axis=1),
            jax.lax.slice_in_dim(nt, k, k + 1, axis=1), 0.0)
    sg_ref[...] = sg


@functools.partial(jax.jit, static_argnames=())
def _router(h, W, expert_embeddings, tau):
    grid = (_NUM_TOK // _BLK,)
    sg, idx, fg = pl.pallas_call(
        _router_block,
        grid=grid,
        in_specs=[
            pl.BlockSpec(memory_space=pltpu.SMEM),
            pl.BlockSpec((_BLK, _IN_DIM), lambda i: (i, 0)),
            pl.BlockSpec((_D_E, _IN_DIM), lambda i: (0, 0)),
            pl.BlockSpec((_NUM_EXPERTS, _D_E), lambda i: (0, 0)),
        ],
        out_specs=[
            pl.BlockSpec((_BLK, _NUM_EXPERTS), lambda i: (i, 0)),
            pl.BlockSpec((_BLK, _TOP_K), lambda i: (i, 0)),
            pl.BlockSpec((_BLK, _NUM_EXPERTS), lambda i: (i, 0)),
        ],
        out_shape=[
            jax.ShapeDtypeStruct((_NUM_TOK, _NUM_EXPERTS), jnp.float32),
            jax.ShapeDtypeStruct((_NUM_TOK, _TOP_K), jnp.int32),
            jax.ShapeDtypeStruct((_NUM_TOK, _NUM_EXPERTS), jnp.float32),
        ],
        compiler_params=pltpu.CompilerParams(
            dimension_semantics=("arbitrary",),
        ),
    )(jnp.reshape(tau, (1,)), h, W, expert_embeddings)
    return sg, idx, fg


def kernel(h, W, expert_embeddings, tau):
    return _router(h, W, expert_embeddings, tau)


# speedup vs baseline: 5.3476x; 1.8468x over previous
import functools

import jax
import jax.numpy as jnp
from jax.experimental import pallas as pl
from jax.experimental.pallas import tpu as pltpu

_NUM_TOK = 8192
_IN_DIM = 4096
_NUM_EXPERTS = 64
_D_E = 64
_TOP_K = 8
_BLK = 512


def _router_block(tau_ref, h_ref, w_ref, ee_ref, sg_ref, idx_ref, fg_ref):
    f32 = jnp.float32
    hp = jax.lax.dot_general(
        h_ref[...], w_ref[...], (((1,), (1,)), ((), ())),
        preferred_element_type=f32, precision=jax.lax.Precision.DEFAULT)
    hpt = hp.T
    nrm = jnp.sqrt(jnp.sum(hpt * hpt, axis=0, keepdims=True))
    hnt = hpt / jnp.maximum(nrm, 1e-12)
    ee = ee_ref[...]
    een = ee / jnp.maximum(
        jnp.sqrt(jnp.sum(ee * ee, axis=-1, keepdims=True)), 1e-12)
    scores = jax.lax.dot_general(
        een, hnt, (((1,), (0,)), ((), ())),
        preferred_element_type=f32, precision=jax.lax.Precision.DEFAULT)
    x = scores / tau_ref[0]
    m = jnp.max(x, axis=0, keepdims=True)
    ex = jnp.exp(x - m)
    fg = ex / jnp.sum(ex, axis=0, keepdims=True)
    fg_ref[...] = fg.T

    iota_e = jax.lax.broadcasted_iota(jnp.int32, fg.shape, 0)
    iota_k = jax.lax.broadcasted_iota(jnp.int32, (_TOP_K, fg.shape[1]), 0)
    work = fg
    vals = jnp.zeros((_TOP_K, fg.shape[1]), f32)
    idxs = jnp.zeros((_TOP_K, fg.shape[1]), jnp.int32)
    for k in range(_TOP_K):
        v = jnp.max(work, axis=0, keepdims=True)
        i = jnp.min(jnp.where(work == v, iota_e, _NUM_EXPERTS),
                    axis=0, keepdims=True)
        vals = jnp.where(iota_k == k, v, vals)
        idxs = jnp.where(iota_k == k, i, idxs)
        work = jnp.where(iota_e == i, -1.0, work)
    idx_ref[...] = idxs.T

    ev = jnp.exp(vals - jax.lax.slice_in_dim(vals, 0, 1, axis=0))
    nt = ev / jnp.sum(ev, axis=0, keepdims=True)

    sg = jnp.zeros(fg.shape, f32)
    for k in range(_TOP_K):
        sg = jnp.where(
            iota_e == jax.lax.slice_in_dim(idxs, k, k + 1, axis=0),
            jax.lax.slice_in_dim(nt, k, k + 1, axis=0), sg)
    sg_ref[...] = sg.T


@functools.partial(jax.jit, static_argnames=())
def _router(h, W, expert_embeddings, tau):
    grid = (_NUM_TOK // _BLK,)
    sg, idx, fg = pl.pallas_call(
        _router_block,
        grid=grid,
        in_specs=[
            pl.BlockSpec(memory_space=pltpu.SMEM),
            pl.BlockSpec((_BLK, _IN_DIM), lambda i: (i, 0)),
            pl.BlockSpec((_D_E, _IN_DIM), lambda i: (0, 0)),
            pl.BlockSpec((_NUM_EXPERTS, _D_E), lambda i: (0, 0)),
        ],
        out_specs=[
            pl.BlockSpec((_BLK, _NUM_EXPERTS), lambda i: (i, 0)),
            pl.BlockSpec((_BLK, _TOP_K), lambda i: (i, 0)),
            pl.BlockSpec((_BLK, _NUM_EXPERTS), lambda i: (i, 0)),
        ],
        out_shape=[
            jax.ShapeDtypeStruct((_NUM_TOK, _NUM_EXPERTS), jnp.float32),
            jax.ShapeDtypeStruct((_NUM_TOK, _TOP_K), jnp.int32),
            jax.ShapeDtypeStruct((_NUM_TOK, _NUM_EXPERTS), jnp.float32),
        ],
        compiler_params=pltpu.CompilerParams(
            dimension_semantics=("arbitrary",),
        ),
    )(jnp.reshape(tau, (1,)), h, W, expert_embeddings)
    return sg, idx, fg


def kernel(h, W, expert_embeddings, tau):
    return _router(h, W, expert_embeddings, tau)


# speedup vs baseline: 5.6289x; 1.0526x over previous
import functools

import jax
import jax.numpy as jnp
from jax.experimental import pallas as pl
from jax.experimental.pallas import tpu as pltpu

_NUM_TOK = 8192
_IN_DIM = 4096
_NUM_EXPERTS = 64
_D_E = 64
_TOP_K = 8
_BLK = 1024


def _router_block(tau_ref, h_ref, w_ref, ee_ref, sg_ref, idx_ref, fg_ref):
    f32 = jnp.float32
    hp = jax.lax.dot_general(
        h_ref[...], w_ref[...], (((1,), (1,)), ((), ())),
        preferred_element_type=f32, precision=jax.lax.Precision.DEFAULT)
    hpt = hp.T
    nrm = jnp.sqrt(jnp.sum(hpt * hpt, axis=0, keepdims=True))
    hnt = hpt / jnp.maximum(nrm, 1e-12)
    ee = ee_ref[...]
    een = ee / jnp.maximum(
        jnp.sqrt(jnp.sum(ee * ee, axis=-1, keepdims=True)), 1e-12)
    scores = jax.lax.dot_general(
        een, hnt, (((1,), (0,)), ((), ())),
        preferred_element_type=f32, precision=jax.lax.Precision.DEFAULT)
    x = scores / tau_ref[0]
    m = jnp.max(x, axis=0, keepdims=True)
    ex = jnp.exp(x - m)
    fg = ex / jnp.sum(ex, axis=0, keepdims=True)
    fg_ref[...] = fg.T

    iota_e = jax.lax.broadcasted_iota(jnp.int32, fg.shape, 0)
    iota_k = jax.lax.broadcasted_iota(jnp.int32, (_TOP_K, fg.shape[1]), 0)
    work = fg
    vals = jnp.zeros((_TOP_K, fg.shape[1]), f32)
    idxs = jnp.zeros((_TOP_K, fg.shape[1]), jnp.int32)
    for k in range(_TOP_K):
        v = jnp.max(work, axis=0, keepdims=True)
        i = jnp.min(jnp.where(work == v, iota_e, _NUM_EXPERTS),
                    axis=0, keepdims=True)
        vals = jnp.where(iota_k == k, v, vals)
        idxs = jnp.where(iota_k == k, i, idxs)
        work = jnp.where(iota_e == i, -1.0, work)
    idx_ref[...] = idxs.T

    ev = jnp.exp(vals - jax.lax.slice_in_dim(vals, 0, 1, axis=0))
    nt = ev / jnp.sum(ev, axis=0, keepdims=True)

    sg = jnp.zeros(fg.shape, f32)
    for k in range(_TOP_K):
        sg = jnp.where(
            iota_e == jax.lax.slice_in_dim(idxs, k, k + 1, axis=0),
            jax.lax.slice_in_dim(nt, k, k + 1, axis=0), sg)
    sg_ref[...] = sg.T


@functools.partial(jax.jit, static_argnames=())
def _router(h, W, expert_embeddings, tau):
    grid = (_NUM_TOK // _BLK,)
    sg, idx, fg = pl.pallas_call(
        _router_block,
        grid=grid,
        in_specs=[
            pl.BlockSpec(memory_space=pltpu.SMEM),
            pl.BlockSpec((_BLK, _IN_DIM), lambda i: (i, 0)),
            pl.BlockSpec((_D_E, _IN_DIM), lambda i: (0, 0)),
            pl.BlockSpec((_NUM_EXPERTS, _D_E), lambda i: (0, 0)),
        ],
        out_specs=[
            pl.BlockSpec((_BLK, _NUM_EXPERTS), lambda i: (i, 0)),
            pl.BlockSpec((_BLK, _TOP_K), lambda i: (i, 0)),
            pl.BlockSpec((_BLK, _NUM_EXPERTS), lambda i: (i, 0)),
        ],
        out_shape=[
            jax.ShapeDtypeStruct((_NUM_TOK, _NUM_EXPERTS), jnp.float32),
            jax.ShapeDtypeStruct((_NUM_TOK, _TOP_K), jnp.int32),
            jax.ShapeDtypeStruct((_NUM_TOK, _NUM_EXPERTS), jnp.float32),
        ],
        compiler_params=pltpu.CompilerParams(
            dimension_semantics=("arbitrary",),
        ),
    )(jnp.reshape(tau, (1,)), h, W, expert_embeddings)
    return sg, idx, fg


def kernel(h, W, expert_embeddings, tau):
    return _router(h, W, expert_embeddings, tau)
